# baseline (speedup 1.0000x reference)
import functools
import math

import jax
import jax.numpy as jnp
from jax import lax
from jax.experimental import pallas as pl
from jax.experimental.pallas import tpu as pltpu
from jax.experimental.pallas import tpu_sc as plsc

B, C, H, W = 2, 128, 256, 256
HW = H * W
N, C_CAM = 20000, 128
HEADS, DHEAD = 4, 32
HD = HEADS * DHEAD
GAMMA = 0.08
SCALE = 1.0 / math.sqrt(DHEAD)

NTILES = 32
RPT = C // NTILES
NC = 2048
NSTEPS = (N + NC - 1) // NC
NP = NSTEPS * NC
NPAD = ((N + 255) // 256) * 256
HG = H // 8
WG = W // 128

_SC_MESH = plsc.VectorSubcoreMesh(core_axis_name="c", subcore_axis_name="s")
_SC_PARAMS = pltpu.CompilerParams(needs_layout_passes=False)


def _lin_body(ii_ref, jj_ref, lin_ref):
    i = jnp.clip(ii_ref[...], 0, H - 1)
    j = jnp.clip(jj_ref[...], 0, W - 1)
    lin_ref[...] = ((i // 8) * WG + j // 128) * 1024 \
        + (i % 8) * 128 + (j % 128)


def _lin_call(ii3, jj3):
    return pl.pallas_call(
        _lin_body,
        grid=(B,),
        in_specs=[
            pl.BlockSpec((1, 1, N), lambda b: (b, 0, 0)),
            pl.BlockSpec((1, 1, N), lambda b: (b, 0, 0)),
        ],
        out_specs=pl.BlockSpec((1, 1, N), lambda b: (b, 0, 0)),
        out_shape=jax.ShapeDtypeStruct((B, 1, N), jnp.int32),
    )(ii3, jj3)


def _sc_gather_body(b, lidar_hbm, lin_hbm, g_hbm, row_v, idx_v, out_v):
    cid = lax.axis_index("c")
    sid = lax.axis_index("s")
    wid = sid * 2 + cid
    pltpu.sync_copy(lin_hbm.at[pl.ds(b * N, N)], idx_v)

    for k in range(RPT):
        r = wid * RPT + k
        pltpu.sync_copy(lidar_hbm.at[pl.ds((b * C + r) * HW, HW)], row_v)

        @plsc.parallel_loop(0, N // 32, unroll=4)
        def _(i):
            off = i * 32
            ix0 = idx_v[pl.ds(off, 16)]
            ix1 = idx_v[pl.ds(off + 16, 16)]
            ga = plsc.load_gather(row_v, [ix0])
            gb = plsc.load_gather(row_v, [ix1])
            out_v[pl.ds(off, 32)] = plsc.pack(
                ga, gb, format=plsc.PackFormat.INTERLEAVED)

        pltpu.sync_copy(out_v, g_hbm.at[pl.ds(r * NPAD, NPAD)])


def _run_sc_gather(b, lidar_phys, lin_flat):
    fn = pl.kernel(
        functools.partial(_sc_gather_body, b),
        out_type=jax.ShapeDtypeStruct((C * NPAD,), jnp.bfloat16),
        mesh=_SC_MESH,
        compiler_params=_SC_PARAMS,
        scratch_types=[
            pltpu.VMEM((HW,), jnp.float32),
            pltpu.VMEM((N,), jnp.int32),
            pltpu.VMEM((NPAD,), jnp.bfloat16),
        ],
    )
    return fn(lidar_phys, lin_flat)


def _sc_scatter_body(b, ot_hbm, lin_hbm, delta_hbm, hits_hbm,
                     acc_v, idx_v, dat_v):
    cid = lax.axis_index("c")
    sid = lax.axis_index("s")
    wid = sid * 2 + cid
    pltpu.sync_copy(lin_hbm.at[pl.ds(b * N, N)], idx_v)
    zero16 = jnp.zeros((16,), jnp.float32)
    ones16 = jnp.ones((16,), jnp.float32)

    def scatter_add_loop():
        @plsc.parallel_loop(0, N // 32, unroll=4)
        def _(i):
            off = i * 32
            dd = dat_v[pl.ds(off, 32)]
            d0, d1 = plsc.unpack(dd, format=plsc.PackFormat.INTERLEAVED)
            ix0 = idx_v[pl.ds(off, 16)]
            ix1 = idx_v[pl.ds(off + 16, 16)]
            plsc.addupdate_scatter(acc_v, [ix0], d0)
            plsc.addupdate_scatter(acc_v, [ix1], d1)

    def scatter_zero_loop():
        @plsc.parallel_loop(0, N // 16, unroll=8)
        def _(i):
            ix = idx_v[pl.ds(i * 16, 16)]
            plsc.store_scatter(acc_v, [ix], zero16)

    @pl.when(wid == 0)
    def _():
        @plsc.parallel_loop(0, HW // 16, unroll=8)
        def _(i):
            acc_v[pl.ds(i * 16, 16)] = zero16

        @plsc.parallel_loop(0, N // 16, unroll=8)
        def _(i):
            ix = idx_v[pl.ds(i * 16, 16)]
            plsc.addupdate_scatter(acc_v, [ix], ones16)

        pltpu.sync_copy(acc_v, hits_hbm)

    for k in range(RPT):
        r = wid * RPT + k
        pltpu.sync_copy(ot_hbm.at[pl.ds(r * NPAD, NPAD)], dat_v)
        scatter_zero_loop()
        scatter_add_loop()
        pltpu.sync_copy(acc_v, delta_hbm.at[pl.ds(r * HW, HW)])


def _run_sc_scatter(b, ot_flat, lin_flat):
    fn = pl.kernel(
        functools.partial(_sc_scatter_body, b),
        out_type=(
            jax.ShapeDtypeStruct((C * HW,), jnp.float32),
            jax.ShapeDtypeStruct((HW,), jnp.float32),
        ),
        mesh=_SC_MESH,
        compiler_params=_SC_PARAMS,
        scratch_types=[
            pltpu.VMEM((HW,), jnp.float32),
            pltpu.VMEM((N,), jnp.int32),
            pltpu.VMEM((NPAD,), jnp.bfloat16),
        ],
    )
    return fn(ot_flat, lin_flat)


def _head_onehot():
    col = lax.broadcasted_iota(jnp.int32, (HEADS, HD), 1) // DHEAD
    row = lax.broadcasted_iota(jnp.int32, (HEADS, HD), 0)
    return (col == row).astype(jnp.float32)


def _tc1_body(g_ref, tok_ref, wq_ref, bq_ref, wk_ref, s_ref):
    g = g_ref[0].astype(jnp.float32)
    tok = tok_ref[0]
    q = jnp.dot(wq_ref[...], g, preferred_element_type=jnp.float32) + bq_ref[...]
    k = lax.dot_general(wk_ref[...], tok, (((1,), (1,)), ((), ())),
                        preferred_element_type=jnp.float32)
    s = jnp.dot(_head_onehot(), q * k, preferred_element_type=jnp.float32)
    s_ref[0] = s * SCALE


def _tc1_call(b, g3, tok, Wq, bq2, Wk):
    return pl.pallas_call(
        _tc1_body,
        grid=(NSTEPS,),
        in_specs=[
            pl.BlockSpec((1, C, NC), lambda n: (0, 0, n)),
            pl.BlockSpec((1, NC, C_CAM), lambda n, _b=b: (_b, n, 0)),
            pl.BlockSpec((HD, C), lambda n: (0, 0)),
            pl.BlockSpec((HD, 1), lambda n: (0, 0)),
            pl.BlockSpec((HD, C_CAM), lambda n: (0, 0)),
        ],
        out_specs=pl.BlockSpec((1, HEADS, NC), lambda n: (0, 0, n)),
        out_shape=jax.ShapeDtypeStruct((1, HEADS, NP), jnp.float32),
    )(g3, tok, Wq, bq2, Wk)


def _tc2_body(s_ref, gw_ref, tok_ref, wv_ref, wo_ref, ot_ref, cs_ref, mz_ref):
    nstep = pl.program_id(0)
    lane_full = lax.broadcasted_iota(jnp.int32, (HEADS, NP), 1)

    @pl.when(nstep == 0)
    def _():
        s = s_ref[0]
        sm = jnp.where(lane_full < N, s, -jnp.inf)
        m = jnp.max(sm, axis=-1, keepdims=True)
        e = jnp.where(lane_full < N, jnp.exp(sm - m), 0.0)
        z = jnp.sum(e, axis=-1, keepdims=True)
        mz_ref[0:HEADS, 0:1] = m
        mz_ref[0:HEADS, 1:2] = z

    m = mz_ref[0:HEADS, 0:1]
    z = mz_ref[0:HEADS, 1:2]
    s_blk = s_ref[0, :, pl.ds(nstep * NC, NC)]
    lane = lax.broadcasted_iota(jnp.int32, (HEADS, NC), 1) + nstep * NC
    p = jnp.where(lane < N, jnp.exp(s_blk - m) / z, 0.0) * gw_ref[0]

    tok = tok_ref[0]
    v = lax.dot_general(wv_ref[...], tok, (((1,), (1,)), ((), ())),
                        preferred_element_type=jnp.float32)
    pe = lax.dot_general(_head_onehot(), p, (((0,), (0,)), ((), ())),
                         preferred_element_type=jnp.float32)
    lane2 = lax.broadcasted_iota(jnp.int32, (HD, NC), 1) + nstep * NC
    fused = jnp.where(lane2 < N, pe * v, 0.0)
    ot_ref[0] = jnp.dot(wo_ref[...], fused,
                        preferred_element_type=jnp.float32).astype(jnp.bfloat16)
    cs = jnp.dot(wo_ref[...], jnp.sum(fused, axis=1, keepdims=True),
                 preferred_element_type=jnp.float32)

    @pl.when(nstep == 0)
    def _():
        cs_ref[0] = cs

    @pl.when(nstep > 0)
    def _():
        cs_ref[0] += cs


def _tc2_call(b, s3, tok, gw3, Wv, Wo):
    return pl.pallas_call(
        _tc2_body,
        grid=(NSTEPS,),
        in_specs=[
            pl.BlockSpec((1, HEADS, NP), lambda n: (0, 0, 0)),
            pl.BlockSpec((1, 1, NC), lambda n, _b=b: (_b, 0, n)),
            pl.BlockSpec((1, NC, C_CAM), lambda n, _b=b: (_b, n, 0)),
            pl.BlockSpec((HD, C_CAM), lambda n: (0, 0)),
            pl.BlockSpec((C, HD), lambda n: (0, 0)),
        ],
        out_specs=[
            pl.BlockSpec((1, C, NC), lambda n: (0, 0, n)),
            pl.BlockSpec((1, C, 1), lambda n: (0, 0, 0)),
        ],
        out_shape=[
            jax.ShapeDtypeStruct((1, C, NPAD), jnp.bfloat16),
            jax.ShapeDtypeStruct((1, C, 1), jnp.float32),
        ],
        scratch_shapes=[pltpu.VMEM((8, 128), jnp.float32)],
    )(s3, gw3, tok, Wv, Wo)


def _tiles_to_pixels(x):
    return jnp.concatenate([x[..., g, :, :] for g in range(WG)], axis=-1)


def _combine_body(lid_ref, dl_ref, al_ref, ht_ref, hf_ref, cs_ref,
                  *rest):
    o_ref = rest[-1]
    hits_full = hf_ref[...]
    nhit = jnp.sum((hits_full > 0.0).astype(jnp.float32))
    mean = cs_ref[0].reshape(C, 1, 1) / (nhit + 1e-6)
    d = _tiles_to_pixels(dl_ref[:, 0])
    maskb = _tiles_to_pixels(ht_ref[0]) > 0.0
    dd = jnp.where(maskb[None], d - mean, 0.0)
    o_ref[0] = lid_ref[0] + dd * (al_ref[0] * GAMMA)


def _combine_call(b, lidar4, d_b, alpha4, h_b, cs_b, prev=None):
    in_specs = [
        pl.BlockSpec((1, C, 8, W), lambda p, _b=b: (_b, 0, p, 0)),
        pl.BlockSpec((C, 1, WG, 8, 128), lambda p: (0, p, 0, 0, 0)),
        pl.BlockSpec((1, 1, 8, W), lambda p: (0, 0, p, 0)),
        pl.BlockSpec((1, WG, 8, 128), lambda p: (p, 0, 0, 0)),
        pl.BlockSpec((HG, WG, 8, 128), lambda p: (0, 0, 0, 0)),
        pl.BlockSpec((1, C, 1), lambda p: (0, 0, 0)),
    ]
    args = [lidar4, d_b, alpha4, h_b, h_b, cs_b]
    aliases = {}
    if prev is not None:
        in_specs.append(pl.BlockSpec((1, C, 8, W), lambda p: (0, 0, 0, 0)))
        args.append(prev)
        aliases = {6: 0}
    return pl.pallas_call(
        _combine_body,
        grid=(HG,),
        in_specs=in_specs,
        out_specs=pl.BlockSpec((1, C, 8, W), lambda p, _b=b: (_b, 0, p, 0)),
        out_shape=jax.ShapeDtypeStruct((B, C, H, W), jnp.float32),
        input_output_aliases=aliases,
    )(*args)


def kernel(lidar_bev, cam_bev_tokens, cam_bev_indices, gate_weights,
           range_alpha, Wq, bq, Wk, Wv, Wo):
    lidar_phys = lidar_bev.reshape(B, C, HG, 8, WG, 128) \
        .transpose(0, 1, 2, 4, 3, 5).reshape(B * C * HW)
    ind = cam_bev_indices.astype(jnp.int32)
    ii3 = ind[..., 0].reshape(B, 1, N)
    jj3 = ind[..., 1].reshape(B, 1, N)
    gw3 = gate_weights.reshape(B, 1, N)
    bq2 = bq.reshape(HD, 1)

    lin3 = _lin_call(ii3, jj3)
    lin_eo = lin3.reshape(B, N // 32, 16, 2).transpose(0, 1, 3, 2) \
        .reshape(B * N)

    out = None
    for b in range(B):
        g3 = _run_sc_gather(b, lidar_phys, lin_eo).reshape(1, C, NPAD)
        s3 = _tc1_call(b, g3, cam_bev_tokens, Wq, bq2, Wk)
        ot3, cs_b = _tc2_call(b, s3, cam_bev_tokens, gw3, Wv, Wo)
        delta_b, hits_b = _run_sc_scatter(b, ot3.reshape(C * NPAD), lin_eo)
        out = _combine_call(b, lidar_bev, delta_b.reshape(C, HG, WG, 8, 128),
                            range_alpha, hits_b.reshape(HG, WG, 8, 128),
                            cs_b, out)
    return out
